# baseline (speedup 1.0000x reference)
import jax
import jax.numpy as jnp
from jax.experimental import pallas as pl

E = 8
C = 512
D_IN = 2048
D_OUT = 1024
B = 1
S = 2048
T = B * S
EPB = 1
STEPS = E // EPB


MCH = 4
MB = T // MCH


def _fused_kernel(x_ref, cw_ref, w_ref, b_ref, out_ref):
    i = pl.program_id(0)
    x = x_ref[0].astype(jnp.bfloat16)
    w = w_ref[0].astype(jnp.bfloat16)
    tmp = jnp.dot(x, w, preferred_element_type=jnp.float32)
    tmp = (tmp + b_ref[0]).astype(jnp.bfloat16)
    for m in range(MCH):
        cw_m = cw_ref[m * MB:(m + 1) * MB, :].astype(jnp.bfloat16)
        part = jnp.dot(cw_m, tmp, preferred_element_type=jnp.float32)
        sl = pl.ds(m * MB, MB)

        @pl.when(i == 0)
        def _init(part=part, sl=sl):
            out_ref[sl, :] = part

        @pl.when(i != 0)
        def _acc(part=part, sl=sl):
            out_ref[sl, :] += part


def kernel(inputs, combine_weights, weight, bias):
    b = bias.reshape(E, 1, D_OUT)

    out = pl.pallas_call(
        _fused_kernel,
        grid=(STEPS,),
        in_specs=[
            pl.BlockSpec((EPB, C, D_IN), lambda i: (i, 0, 0)),
            pl.BlockSpec((T, EPB * C), lambda i: (0, i)),
            pl.BlockSpec((EPB, D_IN, D_OUT), lambda i: (i, 0, 0)),
            pl.BlockSpec((EPB, 1, D_OUT), lambda i: (i, 0, 0)),
        ],
        out_specs=pl.BlockSpec((T, D_OUT), lambda i: (0, 0)),
        out_shape=jax.ShapeDtypeStruct((T, D_OUT), jnp.float32),
    )(inputs, combine_weights, weight, b)
    return out.reshape(B, S, D_OUT)


# speedup vs baseline: 1.1135x; 1.1135x over previous
import jax
import jax.numpy as jnp
from jax.experimental import pallas as pl

E = 8
C = 512
D_IN = 2048
D_OUT = 1024
B = 1
S = 2048
T = B * S
EPB = 1
STEPS = E // EPB


MCH = 4
MB = T // MCH


def _fused_kernel(x_ref, cw_ref, w_ref, b_ref, out_ref):
    i = pl.program_id(0)
    x = x_ref[0].astype(jnp.bfloat16)
    w = w_ref[0].astype(jnp.bfloat16)
    tmp = jnp.dot(x, w, preferred_element_type=jnp.float32)
    tmp = (tmp + b_ref[0]).astype(jnp.bfloat16)
    @pl.when(i == 0)
    def _init():
        for m in range(MCH):
            cw_m = cw_ref[m * MB:(m + 1) * MB, :].astype(jnp.bfloat16)
            out_ref[m * MB:(m + 1) * MB, :] = jnp.dot(
                cw_m, tmp, preferred_element_type=jnp.float32)

    @pl.when(i != 0)
    def _acc():
        for m in range(MCH):
            cw_m = cw_ref[m * MB:(m + 1) * MB, :].astype(jnp.bfloat16)
            out_ref[m * MB:(m + 1) * MB, :] += jnp.dot(
                cw_m, tmp, preferred_element_type=jnp.float32)


def kernel(inputs, combine_weights, weight, bias):
    b = bias.reshape(E, 1, D_OUT)

    out = pl.pallas_call(
        _fused_kernel,
        grid=(STEPS,),
        in_specs=[
            pl.BlockSpec((EPB, C, D_IN), lambda i: (i, 0, 0)),
            pl.BlockSpec((T, EPB * C), lambda i: (0, i)),
            pl.BlockSpec((EPB, D_IN, D_OUT), lambda i: (i, 0, 0)),
            pl.BlockSpec((EPB, 1, D_OUT), lambda i: (i, 0, 0)),
        ],
        out_specs=pl.BlockSpec((T, D_OUT), lambda i: (0, 0)),
        out_shape=jax.ShapeDtypeStruct((T, D_OUT), jnp.float32),
    )(inputs, combine_weights, weight, b)
    return out.reshape(B, S, D_OUT)
